# baseline (speedup 1.0000x reference)
import functools

import jax
import jax.numpy as jnp
from jax import lax
from jax.experimental import pallas as pl
from jax.experimental.pallas import tpu as pltpu
from jax.experimental.pallas import tpu_sc as plsc

N_NODES = 10000
NPAD = 10112
D_FEAT = 128
HID = 16
LSTM_H = 32
N_EDGES = 320000
NC, NS = 2, 16
NW = NC * NS
EDGE_B = 128
EPT = 10240
CHUNKS = EPT // EDGE_B
E_PAD = NW * EPT
ROWS_PT = NPAD // NS

_mesh = plsc.VectorSubcoreMesh(core_axis_name="c", subcore_axis_name="s")

_FULL_OFFS = [0, 128, 256, 384]
_TAIL_OFF, _TAIL = 512, 120


@functools.partial(
    pl.kernel, mesh=_mesh,
    out_type=jax.ShapeDtypeStruct((NC, NPAD, HID), jnp.float32),
    scratch_types=[
        pltpu.VMEM((CHUNKS, EDGE_B), jnp.int32),
        pltpu.VMEM((EDGE_B, HID), jnp.float32),
        pltpu.VMEM_SHARED((NPAD, HID), jnp.float32),
    ],
)
def _sc_degree(dst_hbm, ones_hbm, zeros_hbm, out_hbm, dst_v, ones_v, accum):
    cid = lax.axis_index("c")
    sid = lax.axis_index("s")
    r0 = sid * ROWS_PT
    pltpu.sync_copy(zeros_hbm, accum.at[pl.ds(r0, ROWS_PT)])
    pltpu.sync_copy(ones_hbm, ones_v)
    pltpu.sync_copy(dst_hbm.at[cid, sid], dst_v)
    plsc.subcore_barrier()

    def body(j, carry):
        pltpu.sync_copy(ones_v, accum.at[dst_v.at[j]], add=True)
        return carry

    lax.fori_loop(0, CHUNKS, body, 0)
    plsc.subcore_barrier()
    pltpu.sync_copy(accum.at[pl.ds(r0, ROWS_PT)],
                    out_hbm.at[cid, pl.ds(r0, ROWS_PT)])


@functools.partial(
    pl.kernel, mesh=_mesh,
    out_type=jax.ShapeDtypeStruct((NC, NPAD, HID), jnp.float32),
    scratch_types=[
        pltpu.VMEM((CHUNKS, EDGE_B), jnp.int32),
        pltpu.VMEM((CHUNKS, EDGE_B), jnp.int32),
        pltpu.VMEM((EDGE_B, HID), jnp.float32),
        pltpu.VMEM_SHARED((NPAD, HID), jnp.float32),
        pltpu.VMEM_SHARED((NPAD, HID), jnp.float32),
        pltpu.SemaphoreType.DMA,
    ],
)
def _sc_propagate(y_hbm, src_hbm, dst_hbm, zeros_hbm, out_hbm, src_v, dst_v,
                  rows0, accum, y_s, sem0):
    cid = lax.axis_index("c")
    sid = lax.axis_index("s")
    r0 = sid * ROWS_PT
    pltpu.sync_copy(y_hbm.at[pl.ds(r0, ROWS_PT)], y_s.at[pl.ds(r0, ROWS_PT)])
    pltpu.sync_copy(zeros_hbm, accum.at[pl.ds(r0, ROWS_PT)])
    pltpu.sync_copy(src_hbm.at[cid, sid], src_v)
    pltpu.sync_copy(dst_hbm.at[cid, sid], dst_v)
    plsc.subcore_barrier()

    def body(j, carry):
        pltpu.async_copy(y_s.at[src_v.at[j]], rows0, sem0).wait()
        pltpu.sync_copy(rows0, accum.at[dst_v.at[j]], add=True)
        return carry

    lax.fori_loop(0, CHUNKS, body, 0)
    plsc.subcore_barrier()
    pltpu.sync_copy(accum.at[pl.ds(r0, ROWS_PT)],
                    out_hbm.at[cid, pl.ds(r0, ROWS_PT)])


_GRID = 4
_R = NPAD // _GRID


def _rows(shape):
    return pl.BlockSpec((_R,) + shape[1:], lambda i: (i,) + (0,) * (len(shape) - 1))


def _srows():
    return pl.BlockSpec((NC, _R, HID), lambda i: (0, i, 0))


def _full(shape):
    return pl.BlockSpec(shape, lambda i: (0,) * len(shape))


def _tc_a_body(x_ref, w1_ref, s0_ref, y1_ref, dinv_ref):
    cnt = s0_ref[0] + s0_ref[1]
    deg = cnt[:, 0:1] + 1.0
    dinvb = jnp.broadcast_to(lax.rsqrt(deg), (_R, HID))
    z1 = jnp.dot(x_ref[...], w1_ref[...], preferred_element_type=jnp.float32)
    y1_ref[...] = z1 * dinvb
    dinv_ref[...] = dinvb


def _tc_a(x_pad, W1, s0):
    return pl.pallas_call(
        _tc_a_body,
        grid=(_GRID,),
        in_specs=[_rows((NPAD, D_FEAT)), _full((D_FEAT, HID)), _srows()],
        out_specs=[_rows((NPAD, HID)), _rows((NPAD, HID))],
        out_shape=[jax.ShapeDtypeStruct((NPAD, HID), jnp.float32),
                   jax.ShapeDtypeStruct((NPAD, HID), jnp.float32)],
    )(x_pad, W1, s0)


def _tc_b_body(s1_ref, y1_ref, dinv_ref, b1_ref, w2_ref, x1_ref, y2_ref):
    dinvb = dinv_ref[...]
    p1 = (s1_ref[0] + s1_ref[1] + y1_ref[...]) * dinvb
    x1 = jax.nn.relu(p1 + b1_ref[...])
    z2 = jnp.dot(x1, w2_ref[...], preferred_element_type=jnp.float32)
    x1_ref[...] = x1
    y2_ref[...] = z2 * dinvb


def _tc_b(s1, y1, dinvb, b1, W2):
    return pl.pallas_call(
        _tc_b_body,
        grid=(_GRID,),
        in_specs=[_srows(), _rows((NPAD, HID)), _rows((NPAD, HID)),
                  _full((1, HID)), _full((HID, HID))],
        out_specs=[_rows((NPAD, HID)), _rows((NPAD, HID))],
        out_shape=[jax.ShapeDtypeStruct((NPAD, HID), jnp.float32),
                   jax.ShapeDtypeStruct((NPAD, HID), jnp.float32)],
    )(s1, y1, dinvb, b1, W2)


def _lstm_step(x_t, h, c, wih_t, whh_t, b):
    g = jnp.dot(x_t, wih_t, preferred_element_type=jnp.float32) + b
    if h is not None:
        g = g + jnp.dot(h, whh_t, preferred_element_type=jnp.float32)
    i, f, gg, o = jnp.split(g, 4, axis=-1)
    i = jax.nn.sigmoid(i)
    f = jax.nn.sigmoid(f)
    gg = jnp.tanh(gg)
    o = jax.nn.sigmoid(o)
    c_new = gg * i if c is None else f * c + i * gg
    h_new = o * jnp.tanh(c_new)
    return h_new, c_new


def _tc_c_body(s2_ref, y2_ref, dinv_ref, b2_ref, x1_ref, wihf_ref, whhf_ref,
               bihf_ref, bhhf_ref, wihb_ref, whhb_ref, bihb_ref, bhhb_ref,
               watt_ref, batt_ref, y3_ref):
    dinvb = dinv_ref[...]
    x1 = x1_ref[...]
    x2 = jax.nn.relu((s2_ref[0] + s2_ref[1] + y2_ref[...]) * dinvb
                     + b2_ref[...])
    bf = bihf_ref[...] + bhhf_ref[...]
    bb = bihb_ref[...] + bhhb_ref[...]
    wihf, whhf = wihf_ref[...], whhf_ref[...]
    wihb, whhb = wihb_ref[...], whhb_ref[...]
    h1f, c1f = _lstm_step(x1, None, None, wihf, whhf, bf)
    h2f, _ = _lstm_step(x2, h1f, c1f, wihf, whhf, bf)
    h1b, c1b = _lstm_step(x2, None, None, wihb, whhb, bb)
    h2b, _ = _lstm_step(x1, h1b, c1b, wihb, whhb, bb)
    wa = watt_ref[...]
    wa_f, wa_b = wa[:, :LSTM_H], wa[:, LSTM_H:]
    ba = batt_ref[...]
    a0 = (jnp.sum(h1f * wa_f, axis=1, keepdims=True)
          + jnp.sum(h2b * wa_b, axis=1, keepdims=True) + ba)
    a1 = (jnp.sum(h2f * wa_f, axis=1, keepdims=True)
          + jnp.sum(h1b * wa_b, axis=1, keepdims=True) + ba)
    m = jnp.maximum(a0, a1)
    e0 = jnp.exp(a0 - m)
    e1 = jnp.exp(a1 - m)
    xjk = (e0 * x1 + e1 * x2) / (e0 + e1)
    y3_ref[...] = xjk * dinvb


def _tc_c(s2, y2, dinvb, b2, x1, wihf_t, whhf_t, bihf, bhhf, wihb_t, whhb_t,
          bihb, bhhb, W_att, batt):
    G4 = 4 * LSTM_H
    return pl.pallas_call(
        _tc_c_body,
        grid=(_GRID,),
        in_specs=[_srows(), _rows((NPAD, HID)), _rows((NPAD, HID)),
                  _full((1, HID)), _rows((NPAD, HID)),
                  _full((HID, G4)), _full((LSTM_H, G4)),
                  _full((1, G4)), _full((1, G4)),
                  _full((HID, G4)), _full((LSTM_H, G4)),
                  _full((1, G4)), _full((1, G4)),
                  _full((1, 2 * LSTM_H)), _full((1, 1))],
        out_specs=[_rows((NPAD, HID))],
        out_shape=[jax.ShapeDtypeStruct((NPAD, HID), jnp.float32)],
    )(s2, y2, dinvb, b2, x1, wihf_t, whhf_t, bihf, bhhf, wihb_t, whhb_t,
      bihb, bhhb, W_att, batt)[0]


def _tc_d_body(s3_ref, y3_ref, dinv_ref, wlin_ref, blin_ref, out_ref):
    xp = (s3_ref[0] + s3_ref[1] + y3_ref[...]) * dinv_ref[...]
    o = jnp.dot(xp, wlin_ref[...], preferred_element_type=jnp.float32)
    o = o + blin_ref[...]
    m = jnp.max(o, axis=1, keepdims=True)
    l = o - m
    lse = jnp.log(jnp.sum(jnp.exp(l), axis=1, keepdims=True))
    out_ref[...] = l - lse


def _tc_d(s3, y3, dinvb, W_lin, b_lin):
    return pl.pallas_call(
        _tc_d_body,
        grid=(_GRID,),
        in_specs=[_srows(), _rows((NPAD, HID)), _rows((NPAD, HID)),
                  _full((HID, HID)), _full((1, HID))],
        out_specs=[_rows((NPAD, HID))],
        out_shape=[jax.ShapeDtypeStruct((NPAD, HID), jnp.float32)],
    )(s3, y3, dinvb, W_lin, b_lin)[0]


def kernel(x, edge_index, W1, b1, W2, b2, W_ih_f, W_hh_f, b_ih_f, b_hh_f,
           W_ih_b, W_hh_b, b_ih_b, b_hh_b, W_att, b_att, W_lin, b_lin):
    n = x.shape[0]
    pad = E_PAD - N_EDGES
    src = jnp.concatenate([edge_index[0],
                           jnp.full((pad,), N_NODES, jnp.int32)])
    dst = jnp.concatenate([edge_index[1],
                           jnp.full((pad,), N_NODES, jnp.int32)])
    src4 = src.reshape(NC, NS, CHUNKS, EDGE_B)
    dst4 = dst.reshape(NC, NS, CHUNKS, EDGE_B)
    x_pad = jnp.concatenate(
        [x, jnp.zeros((NPAD - n, D_FEAT), jnp.float32)])
    ones_rows = jnp.ones((EDGE_B, HID), jnp.float32)
    zeros_rows = jnp.zeros((ROWS_PT, HID), jnp.float32)

    s0 = _sc_degree(dst4, ones_rows, zeros_rows)
    y1, dinvb = _tc_a(x_pad, W1, s0)
    s1 = _sc_propagate(y1, src4, dst4, zeros_rows)
    x1, y2 = _tc_b(s1, y1, dinvb, b1.reshape(1, HID), W2)
    s2 = _sc_propagate(y2, src4, dst4, zeros_rows)
    y3 = _tc_c(s2, y2, dinvb, b2.reshape(1, HID), x1,
               W_ih_f.T, W_hh_f.T, b_ih_f.reshape(1, -1),
               b_hh_f.reshape(1, -1), W_ih_b.T, W_hh_b.T,
               b_ih_b.reshape(1, -1), b_hh_b.reshape(1, -1),
               W_att, b_att.reshape(1, 1))
    s3 = _sc_propagate(y3, src4, dst4, zeros_rows)
    out = _tc_d(s3, y3, dinvb, W_lin, b_lin.reshape(1, HID))
    return out[:n]


# speedup vs baseline: 1.0073x; 1.0073x over previous
import functools

import jax
import jax.numpy as jnp
from jax import lax
from jax.experimental import pallas as pl
from jax.experimental.pallas import tpu as pltpu
from jax.experimental.pallas import tpu_sc as plsc

N_NODES = 10000
NPAD = 10112
D_FEAT = 128
HID = 16
LSTM_H = 32
N_EDGES = 320000
NC, NS = 2, 16
NW = NC * NS
EDGE_B = 128
EPT = 10240
CHUNKS = EPT // EDGE_B
E_PAD = NW * EPT
ROWS_PT = NPAD // NS

_mesh = plsc.VectorSubcoreMesh(core_axis_name="c", subcore_axis_name="s")

_INIT_OFFS = [0, 128, 256, 384, 504]
_NZC = len(_INIT_OFFS)


@functools.partial(
    pl.kernel, mesh=_mesh,
    out_type=jax.ShapeDtypeStruct((NC, NPAD, HID), jnp.float32),
    scratch_types=[
        pltpu.VMEM((CHUNKS, EDGE_B), jnp.int32),
        pltpu.VMEM((_NZC, EDGE_B), jnp.int32),
        pltpu.VMEM((EDGE_B, HID), jnp.float32),
        pltpu.VMEM((EDGE_B, HID), jnp.float32),
        pltpu.VMEM_SHARED((NPAD, HID), jnp.float32),
    ],
)
def _sc_degree(dst_hbm, zidx_hbm, ones_hbm, zeros_hbm, out_hbm, dst_v,
               zidx_v, ones_v, zbuf, accum):
    cid = lax.axis_index("c")
    sid = lax.axis_index("s")
    r0 = sid * ROWS_PT
    pltpu.sync_copy(zidx_hbm.at[sid], zidx_v)
    pltpu.sync_copy(zeros_hbm, zbuf)
    for k in range(_NZC):
        pltpu.sync_copy(zbuf, accum.at[zidx_v.at[k]])
    pltpu.sync_copy(ones_hbm, ones_v)
    pltpu.sync_copy(dst_hbm.at[cid, sid], dst_v)
    plsc.subcore_barrier()

    def body(j, carry):
        pltpu.sync_copy(ones_v, accum.at[dst_v.at[j]], add=True)
        return carry

    lax.fori_loop(0, CHUNKS, body, 0)
    plsc.subcore_barrier()
    pltpu.sync_copy(accum.at[pl.ds(r0, ROWS_PT)],
                    out_hbm.at[cid, pl.ds(r0, ROWS_PT)])


@functools.partial(
    pl.kernel, mesh=_mesh,
    out_type=jax.ShapeDtypeStruct((NC, NPAD, HID), jnp.float32),
    scratch_types=[
        pltpu.VMEM((CHUNKS, EDGE_B), jnp.int32),
        pltpu.VMEM((CHUNKS, EDGE_B), jnp.int32),
        pltpu.VMEM((_NZC, EDGE_B), jnp.int32),
        pltpu.VMEM((EDGE_B, HID), jnp.float32),
        pltpu.VMEM_SHARED((NPAD, HID), jnp.float32),
        pltpu.VMEM_SHARED((NPAD, HID), jnp.float32),
        pltpu.SemaphoreType.DMA,
    ],
)
def _sc_propagate(y_hbm, src_hbm, dst_hbm, zidx_hbm, zeros_hbm, out_hbm,
                  src_v, dst_v, zidx_v, rows0, accum, y_s, sem0):
    cid = lax.axis_index("c")
    sid = lax.axis_index("s")
    r0 = sid * ROWS_PT
    pltpu.sync_copy(zidx_hbm.at[sid], zidx_v)
    for k, off in enumerate(_INIT_OFFS):
        pltpu.sync_copy(y_hbm.at[pl.ds(r0 + off, EDGE_B)], rows0)
        pltpu.sync_copy(rows0, y_s.at[zidx_v.at[k]])
    pltpu.sync_copy(zeros_hbm, rows0)
    for k in range(_NZC):
        pltpu.sync_copy(rows0, accum.at[zidx_v.at[k]])
    pltpu.sync_copy(src_hbm.at[cid, sid], src_v)
    pltpu.sync_copy(dst_hbm.at[cid, sid], dst_v)
    plsc.subcore_barrier()

    def body(j, carry):
        pltpu.async_copy(y_s.at[src_v.at[j]], rows0, sem0).wait()
        pltpu.sync_copy(rows0, accum.at[dst_v.at[j]], add=True)
        return carry

    lax.fori_loop(0, CHUNKS, body, 0)
    plsc.subcore_barrier()
    pltpu.sync_copy(accum.at[pl.ds(r0, ROWS_PT)],
                    out_hbm.at[cid, pl.ds(r0, ROWS_PT)])


_GRID = 4
_R = NPAD // _GRID


def _rows(shape):
    return pl.BlockSpec((_R,) + shape[1:], lambda i: (i,) + (0,) * (len(shape) - 1))


def _srows():
    return pl.BlockSpec((NC, _R, HID), lambda i: (0, i, 0))


def _full(shape):
    return pl.BlockSpec(shape, lambda i: (0,) * len(shape))


def _tc_a_body(x_ref, w1_ref, s0_ref, y1_ref, dinv_ref):
    cnt = s0_ref[0] + s0_ref[1]
    deg = cnt[:, 0:1] + 1.0
    dinvb = jnp.broadcast_to(lax.rsqrt(deg), (_R, HID))
    z1 = jnp.dot(x_ref[...], w1_ref[...], preferred_element_type=jnp.float32)
    y1_ref[...] = z1 * dinvb
    dinv_ref[...] = dinvb


def _tc_a(x_pad, W1, s0):
    return pl.pallas_call(
        _tc_a_body,
        grid=(_GRID,),
        in_specs=[_rows((NPAD, D_FEAT)), _full((D_FEAT, HID)), _srows()],
        out_specs=[_rows((NPAD, HID)), _rows((NPAD, HID))],
        out_shape=[jax.ShapeDtypeStruct((NPAD, HID), jnp.float32),
                   jax.ShapeDtypeStruct((NPAD, HID), jnp.float32)],
    )(x_pad, W1, s0)


def _tc_b_body(s1_ref, y1_ref, dinv_ref, b1_ref, w2_ref, x1_ref, y2_ref):
    dinvb = dinv_ref[...]
    p1 = (s1_ref[0] + s1_ref[1] + y1_ref[...]) * dinvb
    x1 = jax.nn.relu(p1 + b1_ref[...])
    z2 = jnp.dot(x1, w2_ref[...], preferred_element_type=jnp.float32)
    x1_ref[...] = x1
    y2_ref[...] = z2 * dinvb


def _tc_b(s1, y1, dinvb, b1, W2):
    return pl.pallas_call(
        _tc_b_body,
        grid=(_GRID,),
        in_specs=[_srows(), _rows((NPAD, HID)), _rows((NPAD, HID)),
                  _full((1, HID)), _full((HID, HID))],
        out_specs=[_rows((NPAD, HID)), _rows((NPAD, HID))],
        out_shape=[jax.ShapeDtypeStruct((NPAD, HID), jnp.float32),
                   jax.ShapeDtypeStruct((NPAD, HID), jnp.float32)],
    )(s1, y1, dinvb, b1, W2)


def _lstm_step(x_t, h, c, wih_t, whh_t, b):
    g = jnp.dot(x_t, wih_t, preferred_element_type=jnp.float32) + b
    if h is not None:
        g = g + jnp.dot(h, whh_t, preferred_element_type=jnp.float32)
    i, f, gg, o = jnp.split(g, 4, axis=-1)
    i = jax.nn.sigmoid(i)
    f = jax.nn.sigmoid(f)
    gg = jnp.tanh(gg)
    o = jax.nn.sigmoid(o)
    c_new = gg * i if c is None else f * c + i * gg
    h_new = o * jnp.tanh(c_new)
    return h_new, c_new


def _tc_c_body(s2_ref, y2_ref, dinv_ref, b2_ref, x1_ref, wihf_ref, whhf_ref,
               bihf_ref, bhhf_ref, wihb_ref, whhb_ref, bihb_ref, bhhb_ref,
               watt_ref, batt_ref, y3_ref):
    dinvb = dinv_ref[...]
    x1 = x1_ref[...]
    x2 = jax.nn.relu((s2_ref[0] + s2_ref[1] + y2_ref[...]) * dinvb
                     + b2_ref[...])
    bf = bihf_ref[...] + bhhf_ref[...]
    bb = bihb_ref[...] + bhhb_ref[...]
    wihf, whhf = wihf_ref[...], whhf_ref[...]
    wihb, whhb = wihb_ref[...], whhb_ref[...]
    h1f, c1f = _lstm_step(x1, None, None, wihf, whhf, bf)
    h2f, _ = _lstm_step(x2, h1f, c1f, wihf, whhf, bf)
    h1b, c1b = _lstm_step(x2, None, None, wihb, whhb, bb)
    h2b, _ = _lstm_step(x1, h1b, c1b, wihb, whhb, bb)
    wa = watt_ref[...]
    wa_f, wa_b = wa[:, :LSTM_H], wa[:, LSTM_H:]
    ba = batt_ref[...]
    a0 = (jnp.sum(h1f * wa_f, axis=1, keepdims=True)
          + jnp.sum(h2b * wa_b, axis=1, keepdims=True) + ba)
    a1 = (jnp.sum(h2f * wa_f, axis=1, keepdims=True)
          + jnp.sum(h1b * wa_b, axis=1, keepdims=True) + ba)
    m = jnp.maximum(a0, a1)
    e0 = jnp.exp(a0 - m)
    e1 = jnp.exp(a1 - m)
    xjk = (e0 * x1 + e1 * x2) / (e0 + e1)
    y3_ref[...] = xjk * dinvb


def _tc_c(s2, y2, dinvb, b2, x1, wihf_t, whhf_t, bihf, bhhf, wihb_t, whhb_t,
          bihb, bhhb, W_att, batt):
    G4 = 4 * LSTM_H
    return pl.pallas_call(
        _tc_c_body,
        grid=(_GRID,),
        in_specs=[_srows(), _rows((NPAD, HID)), _rows((NPAD, HID)),
                  _full((1, HID)), _rows((NPAD, HID)),
                  _full((HID, G4)), _full((LSTM_H, G4)),
                  _full((1, G4)), _full((1, G4)),
                  _full((HID, G4)), _full((LSTM_H, G4)),
                  _full((1, G4)), _full((1, G4)),
                  _full((1, 2 * LSTM_H)), _full((1, 1))],
        out_specs=[_rows((NPAD, HID))],
        out_shape=[jax.ShapeDtypeStruct((NPAD, HID), jnp.float32)],
    )(s2, y2, dinvb, b2, x1, wihf_t, whhf_t, bihf, bhhf, wihb_t, whhb_t,
      bihb, bhhb, W_att, batt)[0]


def _tc_d_body(s3_ref, y3_ref, dinv_ref, wlin_ref, blin_ref, out_ref):
    xp = (s3_ref[0] + s3_ref[1] + y3_ref[...]) * dinv_ref[...]
    o = jnp.dot(xp, wlin_ref[...], preferred_element_type=jnp.float32)
    o = o + blin_ref[...]
    m = jnp.max(o, axis=1, keepdims=True)
    l = o - m
    lse = jnp.log(jnp.sum(jnp.exp(l), axis=1, keepdims=True))
    out_ref[...] = l - lse


def _tc_d(s3, y3, dinvb, W_lin, b_lin):
    return pl.pallas_call(
        _tc_d_body,
        grid=(_GRID,),
        in_specs=[_srows(), _rows((NPAD, HID)), _rows((NPAD, HID)),
                  _full((HID, HID)), _full((1, HID))],
        out_specs=[_rows((NPAD, HID))],
        out_shape=[jax.ShapeDtypeStruct((NPAD, HID), jnp.float32)],
    )(s3, y3, dinvb, W_lin, b_lin)[0]


def kernel(x, edge_index, W1, b1, W2, b2, W_ih_f, W_hh_f, b_ih_f, b_hh_f,
           W_ih_b, W_hh_b, b_ih_b, b_hh_b, W_att, b_att, W_lin, b_lin):
    n = x.shape[0]
    pad = E_PAD - N_EDGES
    src = jnp.concatenate([edge_index[0],
                           jnp.full((pad,), N_NODES, jnp.int32)])
    dst = jnp.concatenate([edge_index[1],
                           jnp.full((pad,), N_NODES, jnp.int32)])
    src4 = src.reshape(NC, NS, CHUNKS, EDGE_B)
    dst4 = dst.reshape(NC, NS, CHUNKS, EDGE_B)
    x_pad = jnp.concatenate(
        [x, jnp.zeros((NPAD - n, D_FEAT), jnp.float32)])
    ones_rows = jnp.ones((EDGE_B, HID), jnp.float32)
    zeros_rows = jnp.zeros((EDGE_B, HID), jnp.float32)
    zidx = (jnp.arange(NS, dtype=jnp.int32)[:, None, None] * ROWS_PT
            + jnp.asarray(_INIT_OFFS, jnp.int32)[None, :, None]
            + jnp.arange(EDGE_B, dtype=jnp.int32)[None, None, :])

    s0 = _sc_degree(dst4, zidx, ones_rows, zeros_rows)
    y1, dinvb = _tc_a(x_pad, W1, s0)
    s1 = _sc_propagate(y1, src4, dst4, zidx, zeros_rows)
    x1, y2 = _tc_b(s1, y1, dinvb, b1.reshape(1, HID), W2)
    s2 = _sc_propagate(y2, src4, dst4, zidx, zeros_rows)
    y3 = _tc_c(s2, y2, dinvb, b2.reshape(1, HID), x1,
               W_ih_f.T, W_hh_f.T, b_ih_f.reshape(1, -1),
               b_hh_f.reshape(1, -1), W_ih_b.T, W_hh_b.T,
               b_ih_b.reshape(1, -1), b_hh_b.reshape(1, -1),
               W_att, b_att.reshape(1, 1))
    s3 = _sc_propagate(y3, src4, dst4, zidx, zeros_rows)
    out = _tc_d(s3, y3, dinvb, W_lin, b_lin.reshape(1, HID))
    return out[:n]
